# Initial kernel scaffold; baseline (speedup 1.0000x reference)
import jax
import jax.numpy as jnp
from jax.experimental import pallas as pl


def kernel(x, edge_index, Ws, bs, fcW, fcb):
    raise NotImplementedError("write your pallas kernel here")


# speedup vs baseline: 31.7276x; 31.7276x over previous
import functools

import jax
import jax.numpy as jnp
from jax import lax
from jax.experimental import pallas as pl
from jax.experimental.pallas import tpu as pltpu
from jax.experimental.pallas import tpu_sc as plsc

N = 10000
E = 320000
HID = 32
NC = 2
NS = 16
NW = NC * NS
EPT = E // NW
CHUNK = 1000
NBLK = EPT // CHUNK
WCH = 1000
NWCH = N // WCH
NP = 10240
DPS = NP // NS

_MESH = plsc.VectorSubcoreMesh(core_axis_name="c", subcore_axis_name="s")


@functools.partial(
    pl.kernel,
    out_type=jax.ShapeDtypeStruct((NC * NP,), jnp.float32),
    mesh=_MESH,
    scratch_types=[
        pltpu.VMEM_SHARED((NP,), jnp.float32),
        pltpu.VMEM((CHUNK,), jnp.int32),
        pltpu.VMEM((CHUNK,), jnp.float32),
    ],
)
def _degree_hist(dst_hbm, ones_hbm, zeros_hbm, out_hbm, acc, dstv, onesv):
    c = lax.axis_index("c")
    s = lax.axis_index("s")
    wid = c * NS + s
    pltpu.sync_copy(zeros_hbm, acc.at[pl.ds(s * DPS, DPS)])
    pltpu.sync_copy(ones_hbm, onesv)
    plsc.subcore_barrier()
    base = wid * EPT
    for blk in range(NBLK):
        pltpu.sync_copy(dst_hbm.at[pl.ds(base + blk * CHUNK, CHUNK)], dstv)
        pltpu.sync_copy(onesv, acc.at[dstv], add=True)
    plsc.subcore_barrier()
    pltpu.sync_copy(acc.at[pl.ds(s * DPS, DPS)],
                    out_hbm.at[pl.ds(c * NP + s * DPS, DPS)])


@functools.partial(
    pl.kernel,
    out_type=jax.ShapeDtypeStruct((NC * N, HID), jnp.float32),
    mesh=_MESH,
    scratch_types=[
        pltpu.VMEM_SHARED((N, HID), jnp.float32),
        pltpu.VMEM((CHUNK,), jnp.int32),
        pltpu.VMEM((CHUNK,), jnp.int32),
        pltpu.VMEM((CHUNK, HID), jnp.float32),
        pltpu.SemaphoreType.DMA,
    ],
    compiler_params=pltpu.CompilerParams(use_tc_tiling_on_sc=False),
)
def _edge_aggregate(table_hbm, src_hbm, dst_hbm, zrows_hbm, out_hbm,
                    acc, srcv, dstv, rowsv, sem):
    c = lax.axis_index("c")
    s = lax.axis_index("s")
    wid = c * NS + s

    @pl.when(s < NWCH)
    def _zero():
        pltpu.sync_copy(zrows_hbm, acc.at[pl.ds(s * WCH, WCH)])

    plsc.subcore_barrier()
    base = wid * EPT
    for blk in range(NBLK):
        off = base + blk * CHUNK
        pltpu.sync_copy(src_hbm.at[pl.ds(off, CHUNK)], srcv)
        pltpu.sync_copy(dst_hbm.at[pl.ds(off, CHUNK)], dstv)
        pltpu.async_copy(table_hbm.at[srcv], rowsv, sem).wait()
        pltpu.sync_copy(rowsv, acc.at[dstv], add=True)
    plsc.subcore_barrier()

    @pl.when(s < NWCH)
    def _writeout():
        pltpu.sync_copy(acc.at[pl.ds(s * WCH, WCH)],
                        out_hbm.at[pl.ds(c * N + s * WCH, WCH)])


def _prep_body(degp_ref, x_ref, w_ref, dinv_ref, xws_ref):
    deg = degp_ref[0] + degp_ref[1] + 1.0
    dinv = lax.rsqrt(deg)
    dinv_ref[...] = dinv
    xw = jnp.dot(x_ref[...], w_ref[...], preferred_element_type=jnp.float32)
    xws_ref[...] = xw * dinv


def _prep(degp, x, w0):
    return pl.pallas_call(
        _prep_body,
        out_shape=[
            jax.ShapeDtypeStruct((N, 1), jnp.float32),
            jax.ShapeDtypeStruct((N, HID), jnp.float32),
        ],
    )(degp, x, w0)


def _boundary_body(has_jk, p_ref, xws_ref, dinv_ref, b_ref, w_ref, jk_ref,
                   xwsn_ref, jko_ref):
    total = p_ref[0] + p_ref[1] + xws_ref[...]
    h = jnp.maximum(total * dinv_ref[...] + b_ref[...], 0.0)
    jko = jnp.maximum(jk_ref[...], h) if has_jk else h
    jko_ref[...] = jko
    xwsn_ref[...] = jnp.dot(h, w_ref[...],
                            preferred_element_type=jnp.float32) * dinv_ref[...]


def _boundary(partials, xws, dinv, b, w_next, jk):
    has_jk = jk is not None
    args = [partials, xws, dinv, b, w_next] + ([jk] if has_jk else [])
    body = functools.partial(_boundary_body, has_jk)
    if not has_jk:
        def body(p, xw, di, bb, ww, xn, jo):
            _boundary_body(False, p, xw, di, bb, ww, None, xn, jo)
    return pl.pallas_call(
        body,
        out_shape=[
            jax.ShapeDtypeStruct((N, HID), jnp.float32),
            jax.ShapeDtypeStruct((N, HID), jnp.float32),
        ],
    )(*args)


def _final_body(p_ref, xws_ref, dinv_ref, b_ref, jk_ref, fcw_ref, fcb_ref,
                out_ref):
    total = p_ref[0] + p_ref[1] + xws_ref[...]
    h = jnp.maximum(total * dinv_ref[...] + b_ref[...], 0.0)
    jk = jnp.maximum(jk_ref[...], h)
    logits = jnp.dot(jk, fcw_ref[...],
                     preferred_element_type=jnp.float32) + fcb_ref[...]
    m = jnp.max(logits, axis=1, keepdims=True)
    z = logits - m
    lse = jnp.log(jnp.sum(jnp.exp(z), axis=1, keepdims=True))
    out_ref[...] = z - lse


def _final(partials, xws, dinv, b, jk, fcw, fcb):
    nclass = fcw.shape[1]
    return pl.pallas_call(
        _final_body,
        out_shape=jax.ShapeDtypeStruct((N, nclass), jnp.float32),
    )(partials, xws, dinv, b, jk, fcw, fcb)


def kernel(x, edge_index, Ws, bs, fcW, fcb):
    src = edge_index[0]
    dst = edge_index[1]
    ones_c = jnp.ones((CHUNK,), jnp.float32)
    zeros_d = jnp.zeros((DPS,), jnp.float32)
    zeros_r = jnp.zeros((WCH, HID), jnp.float32)

    degp = _degree_hist(dst, ones_c, zeros_d)
    degp = degp.reshape(NC, NP, 1)[:, :N, :]
    dinv, xws = _prep(degp, x, Ws[0])

    jk = None
    nlayers = len(Ws)
    for l in range(nlayers):
        partials = _edge_aggregate(xws, src, dst, zeros_r)
        partials = partials.reshape(NC, N, HID)
        b = bs[l].reshape(1, HID)
        if l + 1 < nlayers:
            xws, jk = _boundary(partials, xws, dinv, b, Ws[l + 1], jk)
        else:
            out = _final(partials, xws, dinv, b, jk, fcW,
                         fcb.reshape(1, -1))
    return out


# speedup vs baseline: 41.1880x; 1.2982x over previous
import functools

import jax
import jax.numpy as jnp
from jax import lax
from jax.experimental import pallas as pl
from jax.experimental.pallas import tpu as pltpu
from jax.experimental.pallas import tpu_sc as plsc

N = 10000
E = 320000
HID = 32
NC = 2
NS = 16
NW = NC * NS
EPT = E // NW
CHUNK = 1000
NBLK = EPT // CHUNK
WCH = 1000
NWCH = N // WCH
NP = 10240
DPS = NP // NS

_MESH = plsc.VectorSubcoreMesh(core_axis_name="c", subcore_axis_name="s")


@functools.partial(
    pl.kernel,
    out_type=jax.ShapeDtypeStruct((NC * NP,), jnp.float32),
    mesh=_MESH,
    scratch_types=[
        pltpu.VMEM_SHARED((NP,), jnp.float32),
        pltpu.VMEM((CHUNK,), jnp.int32),
        pltpu.VMEM((CHUNK,), jnp.float32),
    ],
)
def _degree_hist(dst_hbm, ones_hbm, zeros_hbm, out_hbm, acc, dstv, onesv):
    c = lax.axis_index("c")
    s = lax.axis_index("s")
    wid = c * NS + s
    pltpu.sync_copy(zeros_hbm, acc.at[pl.ds(s * DPS, DPS)])
    pltpu.sync_copy(ones_hbm, onesv)
    plsc.subcore_barrier()
    base = wid * EPT
    for blk in range(NBLK):
        pltpu.sync_copy(dst_hbm.at[pl.ds(base + blk * CHUNK, CHUNK)], dstv)
        pltpu.sync_copy(onesv, acc.at[dstv], add=True)
    plsc.subcore_barrier()
    pltpu.sync_copy(acc.at[pl.ds(s * DPS, DPS)],
                    out_hbm.at[pl.ds(c * NP + s * DPS, DPS)])


@functools.partial(
    pl.kernel,
    out_type=jax.ShapeDtypeStruct((NC * N, HID), jnp.float32),
    mesh=_MESH,
    scratch_types=[
        pltpu.VMEM_SHARED((N, HID), jnp.float32),
        pltpu.VMEM((NBLK, CHUNK), jnp.int32),
        pltpu.VMEM((NBLK, CHUNK), jnp.int32),
        pltpu.VMEM((2, CHUNK, HID), jnp.float32),
        pltpu.SemaphoreType.DMA,
        pltpu.SemaphoreType.DMA,
        pltpu.SemaphoreType.DMA,
    ],
    compiler_params=pltpu.CompilerParams(use_tc_tiling_on_sc=False),
)
def _edge_aggregate(table_hbm, src_hbm, dst_hbm, zrows_hbm, out_hbm,
                    acc, srci, dsti, rowsv, semi, semg0, semg1):
    c = lax.axis_index("c")
    s = lax.axis_index("s")
    wid = c * NS + s
    semg = (semg0, semg1)

    cpi = pltpu.async_copy(src_hbm.at[wid], srci, semi)
    cpd = pltpu.async_copy(dst_hbm.at[wid], dsti, semi)

    @pl.when(s < NWCH)
    def _zero():
        pltpu.sync_copy(zrows_hbm, acc.at[pl.ds(s * WCH, WCH)])

    cpi.wait()
    cpd.wait()
    gathers = [None, None]
    gathers[0] = pltpu.async_copy(table_hbm.at[srci.at[0]], rowsv.at[0],
                                  semg[0])
    plsc.subcore_barrier()
    for blk in range(NBLK):
        p = blk % 2
        if blk + 1 < NBLK:
            gathers[1 - p] = pltpu.async_copy(
                table_hbm.at[srci.at[blk + 1]], rowsv.at[1 - p],
                semg[1 - p])
        gathers[p].wait()
        pltpu.sync_copy(rowsv.at[p], acc.at[dsti.at[blk]], add=True)
    plsc.subcore_barrier()

    @pl.when(s < NWCH)
    def _writeout():
        pltpu.sync_copy(acc.at[pl.ds(s * WCH, WCH)],
                        out_hbm.at[pl.ds(c * N + s * WCH, WCH)])


def _prep_body(degp_ref, x_ref, w_ref, dinv_ref, xws_ref):
    deg = degp_ref[0] + degp_ref[1] + 1.0
    dinv = lax.rsqrt(deg)
    dinv_ref[...] = dinv
    xw = jnp.dot(x_ref[...], w_ref[...], preferred_element_type=jnp.float32)
    xws_ref[...] = xw * dinv


def _prep(degp, x, w0):
    return pl.pallas_call(
        _prep_body,
        out_shape=[
            jax.ShapeDtypeStruct((N, 1), jnp.float32),
            jax.ShapeDtypeStruct((N, HID), jnp.float32),
        ],
    )(degp, x, w0)


def _boundary_body(has_jk, p_ref, xws_ref, dinv_ref, b_ref, w_ref, jk_ref,
                   xwsn_ref, jko_ref):
    total = p_ref[0] + p_ref[1] + xws_ref[...]
    h = jnp.maximum(total * dinv_ref[...] + b_ref[...], 0.0)
    jko = jnp.maximum(jk_ref[...], h) if has_jk else h
    jko_ref[...] = jko
    xwsn_ref[...] = jnp.dot(h, w_ref[...],
                            preferred_element_type=jnp.float32) * dinv_ref[...]


def _boundary(partials, xws, dinv, b, w_next, jk):
    has_jk = jk is not None
    args = [partials, xws, dinv, b, w_next] + ([jk] if has_jk else [])
    body = functools.partial(_boundary_body, has_jk)
    if not has_jk:
        def body(p, xw, di, bb, ww, xn, jo):
            _boundary_body(False, p, xw, di, bb, ww, None, xn, jo)
    return pl.pallas_call(
        body,
        out_shape=[
            jax.ShapeDtypeStruct((N, HID), jnp.float32),
            jax.ShapeDtypeStruct((N, HID), jnp.float32),
        ],
    )(*args)


def _final_body(p_ref, xws_ref, dinv_ref, b_ref, jk_ref, fcw_ref, fcb_ref,
                out_ref):
    total = p_ref[0] + p_ref[1] + xws_ref[...]
    h = jnp.maximum(total * dinv_ref[...] + b_ref[...], 0.0)
    jk = jnp.maximum(jk_ref[...], h)
    logits = jnp.dot(jk, fcw_ref[...],
                     preferred_element_type=jnp.float32) + fcb_ref[...]
    m = jnp.max(logits, axis=1, keepdims=True)
    z = logits - m
    lse = jnp.log(jnp.sum(jnp.exp(z), axis=1, keepdims=True))
    out_ref[...] = z - lse


def _final(partials, xws, dinv, b, jk, fcw, fcb):
    nclass = fcw.shape[1]
    return pl.pallas_call(
        _final_body,
        out_shape=jax.ShapeDtypeStruct((N, nclass), jnp.float32),
    )(partials, xws, dinv, b, jk, fcw, fcb)


def kernel(x, edge_index, Ws, bs, fcW, fcb):
    src = edge_index[0]
    dst = edge_index[1]
    src3 = src.reshape(NW, NBLK, CHUNK)
    dst3 = dst.reshape(NW, NBLK, CHUNK)
    ones_c = jnp.ones((CHUNK,), jnp.float32)
    zeros_d = jnp.zeros((DPS,), jnp.float32)
    zeros_r = jnp.zeros((WCH, HID), jnp.float32)

    degp = _degree_hist(dst, ones_c, zeros_d)
    degp = degp.reshape(NC, NP, 1)[:, :N, :]
    dinv, xws = _prep(degp, x, Ws[0])

    jk = None
    nlayers = len(Ws)
    for l in range(nlayers):
        partials = _edge_aggregate(xws, src3, dst3, zeros_r)
        partials = partials.reshape(NC, N, HID)
        b = bs[l].reshape(1, HID)
        if l + 1 < nlayers:
            xws, jk = _boundary(partials, xws, dinv, b, Ws[l + 1], jk)
        else:
            out = _final(partials, xws, dinv, b, jk, fcW,
                         fcb.reshape(1, -1))
    return out


# speedup vs baseline: 55.3210x; 1.3431x over previous
import functools

import jax
import jax.numpy as jnp
from jax import lax
from jax.experimental import pallas as pl
from jax.experimental.pallas import tpu as pltpu
from jax.experimental.pallas import tpu_sc as plsc

N = 10000
E = 320000
HID = 32
PACK = 4
R4 = N // PACK
PW = PACK * HID
NC = 2
NS = 16
NW = NC * NS
EPT = E // NW
CHUNK = 1000
NBLK = EPT // CHUNK
WCH = 1000
NWCH = N // WCH

_MESH = plsc.VectorSubcoreMesh(core_axis_name="c", subcore_axis_name="s")


@functools.partial(
    pl.kernel,
    out_type=jax.ShapeDtypeStruct((NC * N, HID), jnp.float32),
    mesh=_MESH,
    scratch_types=[
        pltpu.VMEM_SHARED((N, HID), jnp.float32),
        pltpu.VMEM((NBLK, CHUNK), jnp.int32),
        pltpu.VMEM((CHUNK, HID), jnp.float32),
        pltpu.SemaphoreType.DMA,
    ],
    compiler_params=pltpu.CompilerParams(use_tc_tiling_on_sc=False),
)
def _degree_hist(dst_hbm, ones_hbm, zrows_hbm, out_hbm, acc, dsti, onesv,
                 semi):
    c = lax.axis_index("c")
    s = lax.axis_index("s")
    wid = c * NS + s
    cpd = pltpu.async_copy(dst_hbm.at[wid], dsti, semi)
    cpo = pltpu.async_copy(ones_hbm, onesv, semi)

    @pl.when(s < NWCH)
    def _zero():
        pltpu.sync_copy(zrows_hbm, acc.at[pl.ds(s * WCH, WCH)])

    cpd.wait()
    cpo.wait()
    plsc.subcore_barrier()
    for blk in range(NBLK):
        pltpu.sync_copy(onesv, acc.at[dsti.at[blk]], add=True)
    plsc.subcore_barrier()

    @pl.when(s < NWCH)
    def _writeout():
        pltpu.sync_copy(acc.at[pl.ds(s * WCH, WCH)],
                        out_hbm.at[pl.ds(c * N + s * WCH, WCH)])


@functools.partial(
    pl.kernel,
    out_type=jax.ShapeDtypeStruct((NC * N, HID), jnp.float32),
    mesh=_MESH,
    scratch_types=[
        pltpu.VMEM_SHARED((N, HID), jnp.float32),
        pltpu.VMEM((NBLK, CHUNK), jnp.int32),
        pltpu.VMEM((NBLK, CHUNK), jnp.int32),
        pltpu.VMEM((2, CHUNK, HID), jnp.float32),
        pltpu.SemaphoreType.DMA,
        pltpu.SemaphoreType.DMA,
        pltpu.SemaphoreType.DMA,
    ],
    compiler_params=pltpu.CompilerParams(use_tc_tiling_on_sc=False),
)
def _edge_aggregate(table_hbm, src_hbm, dst_hbm, zrows_hbm, out_hbm,
                    acc, srci, dsti, rowsv, semi, semg0, semg1):
    c = lax.axis_index("c")
    s = lax.axis_index("s")
    wid = c * NS + s
    semg = (semg0, semg1)

    cpi = pltpu.async_copy(src_hbm.at[wid], srci, semi)
    cpd = pltpu.async_copy(dst_hbm.at[wid], dsti, semi)

    @pl.when(s < NWCH)
    def _zero():
        pltpu.sync_copy(zrows_hbm, acc.at[pl.ds(s * WCH, WCH)])

    cpi.wait()
    cpd.wait()
    gathers = [None, None]
    gathers[0] = pltpu.async_copy(table_hbm.at[srci.at[0]], rowsv.at[0],
                                  semg[0])
    plsc.subcore_barrier()
    for blk in range(NBLK):
        p = blk % 2
        if blk + 1 < NBLK:
            gathers[1 - p] = pltpu.async_copy(
                table_hbm.at[srci.at[blk + 1]], rowsv.at[1 - p],
                semg[1 - p])
        gathers[p].wait()
        pltpu.sync_copy(rowsv.at[p], acc.at[dsti.at[blk]], add=True)
    plsc.subcore_barrier()

    @pl.when(s < NWCH)
    def _writeout():
        pltpu.sync_copy(acc.at[pl.ds(s * WCH, WCH)],
                        out_hbm.at[pl.ds(c * N + s * WCH, WCH)])


def _prep_body(degp_ref, x4_ref, w14_ref, dinv_ref, xws_ref):
    deg = degp_ref[0] + degp_ref[1] + 1.0
    dinv = lax.rsqrt(deg)
    dinv_ref[...] = dinv
    xw = jnp.dot(x4_ref[...], w14_ref[...], preferred_element_type=jnp.float32)
    xws_ref[...] = xw * dinv


def _prep(degp, x4, w14):
    return pl.pallas_call(
        _prep_body,
        out_shape=[
            jax.ShapeDtypeStruct((R4, PW), jnp.float32),
            jax.ShapeDtypeStruct((R4, PW), jnp.float32),
        ],
    )(degp, x4, w14)


def _boundary_body(has_jk, p_ref, xws_ref, dinv_ref, b_ref, w_ref, jk_ref,
                   xwsn_ref, jko_ref):
    total = p_ref[0] + p_ref[1] + xws_ref[...]
    h = jnp.maximum(total * dinv_ref[...] + b_ref[...], 0.0)
    jko = jnp.maximum(jk_ref[...], h) if has_jk else h
    jko_ref[...] = jko
    xwsn_ref[...] = jnp.dot(h, w_ref[...],
                            preferred_element_type=jnp.float32) * dinv_ref[...]


def _boundary(partials, xws, dinv, b, w_next, jk):
    has_jk = jk is not None
    args = [partials, xws, dinv, b, w_next] + ([jk] if has_jk else [])
    if has_jk:
        body = functools.partial(_boundary_body, True)
    else:
        def body(p, xw, di, bb, ww, xn, jo):
            _boundary_body(False, p, xw, di, bb, ww, None, xn, jo)
    return pl.pallas_call(
        body,
        out_shape=[
            jax.ShapeDtypeStruct((R4, PW), jnp.float32),
            jax.ShapeDtypeStruct((R4, PW), jnp.float32),
        ],
    )(*args)


def _final_body(nclass, p_ref, xws_ref, dinv_ref, b_ref, jk_ref, fcw_ref,
                fcb_ref, out_ref):
    total = p_ref[0] + p_ref[1] + xws_ref[...]
    h = jnp.maximum(total * dinv_ref[...] + b_ref[...], 0.0)
    jk = jnp.maximum(jk_ref[...], h)
    logits = jnp.dot(jk, fcw_ref[...],
                     preferred_element_type=jnp.float32) + fcb_ref[...]
    for j in range(PACK):
        blk = logits[:, j * nclass:(j + 1) * nclass]
        m = jnp.max(blk, axis=1, keepdims=True)
        z = blk - m
        lse = jnp.log(jnp.sum(jnp.exp(z), axis=1, keepdims=True))
        out_ref[:, j * nclass:(j + 1) * nclass] = z - lse


def _final(partials, xws, dinv, b, jk, fcw4, fcb4):
    nclass = fcw4.shape[1] // PACK
    return pl.pallas_call(
        functools.partial(_final_body, nclass),
        out_shape=jax.ShapeDtypeStruct((R4, PACK * nclass), jnp.float32),
    )(partials, xws, dinv, b, jk, fcw4, fcb4)


def _blkdiag(w):
    a, b = w.shape
    out = jnp.zeros((PACK * a, PACK * b), w.dtype)
    for j in range(PACK):
        out = out.at[j * a:(j + 1) * a, j * b:(j + 1) * b].set(w)
    return out


def kernel(x, edge_index, Ws, bs, fcW, fcb):
    src3 = edge_index[0].reshape(NW, NBLK, CHUNK)
    dst3 = edge_index[1].reshape(NW, NBLK, CHUNK)
    ones_r = jnp.ones((CHUNK, HID), jnp.float32)
    zeros_r = jnp.zeros((WCH, HID), jnp.float32)

    w14 = _blkdiag(Ws[0])
    w4s = [_blkdiag(w) for w in Ws[1:]]
    fcw4 = _blkdiag(fcW)
    b4s = [jnp.tile(b, PACK).reshape(1, PW) for b in bs]
    fcb4 = jnp.tile(fcb, PACK).reshape(1, -1)

    deg4 = _degree_hist(dst3, ones_r, zeros_r).reshape(NC, R4, PW)
    x4 = x.reshape(R4, PACK * x.shape[1])
    dinv, xws = _prep(deg4, x4, w14)

    jk = None
    nlayers = len(Ws)
    for l in range(nlayers):
        partials = _edge_aggregate(xws.reshape(N, HID), src3, dst3, zeros_r)
        p4 = partials.reshape(NC, R4, PW)
        if l + 1 < nlayers:
            xws, jk = _boundary(p4, xws, dinv, b4s[l], w4s[l], jk)
        else:
            out4 = _final(p4, xws, dinv, b4s[l], jk, fcw4, fcb4)
    return out4.reshape(N, fcW.shape[1])
